# Initial kernel scaffold; baseline (speedup 1.0000x reference)
import functools

import jax
import jax.numpy as jnp
from jax import lax
from jax.experimental import pallas as pl
from jax.experimental.pallas import tpu as pltpu
from jax.experimental.pallas import tpu_sc as plsc

N = 320000
M = 10000
D = 128

NC = 2
NS = 16
NW = NC * NS

CH = 125
ROWS_PT = N // NW
CPT = ROWS_PT // CH
MS = M // NS

_mesh = plsc.VectorSubcoreMesh(core_axis_name="c", subcore_axis_name="s")


@functools.partial(
    pl.kernel,
    out_type=jax.ShapeDtypeStruct((NC, M, D), jnp.float32),
    mesh=_mesh,
    scratch_types=[
        pltpu.VMEM((CPT, CH), jnp.int32),
        pltpu.VMEM((CH, D), jnp.float32),
        pltpu.VMEM_SHARED((M, D), jnp.float32),
    ],
)
def _scatter_partials(mu_hbm, idx_hbm, zeros_hbm, out_hbm, idxblk, mubuf, acc):
    cid = lax.axis_index("c")
    sid = lax.axis_index("s")
    wid = sid * NC + cid
    row0 = wid * ROWS_PT
    pltpu.sync_copy(idx_hbm.at[pl.ds(wid * CPT, CPT)], idxblk)
    msl = pl.ds(sid * MS, MS)
    pltpu.sync_copy(zeros_hbm.at[msl], acc.at[msl])
    plsc.subcore_barrier()

    def body(j, carry):
        pltpu.sync_copy(mu_hbm.at[pl.ds(row0 + j * CH, CH)], mubuf)
        pltpu.sync_copy(mubuf, acc.at[idxblk.at[j]], add=True)
        return carry

    lax.fori_loop(0, CPT, body, 0)
    plsc.subcore_barrier()
    pltpu.sync_copy(acc.at[msl], out_hbm.at[cid].at[msl])


BM = 1000


def _combine_body(p_ref, obs_ref, gtab_ref, loss_ref):
    i = pl.program_id(0)
    g = p_ref[0] + p_ref[1] - obs_ref[...]
    gtab_ref[...] = g

    @pl.when(i == 0)
    def _():
        loss_ref[0, 0] = 0.0

    loss_ref[0, 0] += 0.5 * jnp.sum(g * g)


_combine = pl.pallas_call(
    _combine_body,
    grid=(M // BM,),
    in_specs=[
        pl.BlockSpec((2, BM, D), lambda i: (0, i, 0)),
        pl.BlockSpec((BM, D), lambda i: (i, 0)),
    ],
    out_specs=[
        pl.BlockSpec((BM, D), lambda i: (i, 0)),
        pl.BlockSpec(memory_space=pltpu.SMEM),
    ],
    out_shape=[
        jax.ShapeDtypeStruct((M, D), jnp.float32),
        jax.ShapeDtypeStruct((1, 1), jnp.float32),
    ],
)


@functools.partial(
    pl.kernel,
    out_type=jax.ShapeDtypeStruct((N, D), jnp.float32),
    mesh=_mesh,
    scratch_types=[
        pltpu.VMEM((CPT, CH), jnp.int32),
        pltpu.VMEM((CH, D), jnp.float32),
        pltpu.SemaphoreType.DMA,
    ],
)
def _gather_grad(gtab_hbm, idx_hbm, out_hbm, idxblk, buf, sem):
    cid = lax.axis_index("c")
    sid = lax.axis_index("s")
    wid = sid * NC + cid
    row0 = wid * ROWS_PT
    pltpu.sync_copy(idx_hbm.at[pl.ds(wid * CPT, CPT)], idxblk)

    def body(j, carry):
        pltpu.async_copy(gtab_hbm.at[idxblk.at[j]], buf, sem).wait()
        pltpu.sync_copy(buf, out_hbm.at[pl.ds(row0 + j * CH, CH)])
        return carry

    lax.fori_loop(0, CPT, body, 0)


def kernel(mu_0, obs, idx):
    idx2d = idx.reshape(NW * CPT, CH)
    zeros = jnp.zeros((M, D), jnp.float32)
    partials = _scatter_partials(mu_0, idx2d, zeros)
    gtab, loss2d = _combine(partials, obs)
    grad = _gather_grad(gtab, idx2d)
    return loss2d[0, 0], grad


# speedup vs baseline: 2.4485x; 2.4485x over previous
import functools

import jax
import jax.numpy as jnp
from jax import lax
from jax.experimental import pallas as pl
from jax.experimental.pallas import tpu as pltpu
from jax.experimental.pallas import tpu_sc as plsc

N = 320000
M = 10000
D = 128

NC = 2
NS = 16
NW = NC * NS

CH = 80
ROWS_PT = N // NW
CPT = ROWS_PT // CH
MSA = 624
MREM = M - NS * MSA

_mesh = plsc.VectorSubcoreMesh(core_axis_name="c", subcore_axis_name="s")


@functools.partial(
    pl.kernel,
    out_type=jax.ShapeDtypeStruct((NC, M, D), jnp.float32),
    mesh=_mesh,
    scratch_types=[
        pltpu.VMEM((CPT, CH), jnp.int32),
        pltpu.VMEM((CH, D), jnp.float32),
        pltpu.VMEM_SHARED((M, D), jnp.float32),
    ],
)
def _scatter_partials(mu_hbm, idx_hbm, zeros_hbm, out_hbm, idxblk, mubuf, acc):
    cid = lax.axis_index("c")
    sid = lax.axis_index("s")
    wid = sid * NC + cid
    row0 = wid * ROWS_PT
    pltpu.sync_copy(idx_hbm.at[wid], idxblk)
    msl = pl.ds(sid * MSA, MSA)
    pltpu.sync_copy(zeros_hbm.at[msl], acc.at[msl])

    @pl.when(sid == 0)
    def _():
        rsl = pl.ds(NS * MSA, MREM)
        pltpu.sync_copy(zeros_hbm.at[rsl], acc.at[rsl])

    plsc.subcore_barrier()

    def body(j, carry):
        pltpu.sync_copy(mu_hbm.at[pl.ds(row0 + j * CH, CH)], mubuf)
        pltpu.sync_copy(mubuf, acc.at[idxblk.at[j]], add=True)
        return carry

    lax.fori_loop(0, CPT, body, 0)
    plsc.subcore_barrier()
    pltpu.sync_copy(acc.at[msl], out_hbm.at[cid].at[msl])

    @pl.when(sid == 0)
    def _():
        rsl = pl.ds(NS * MSA, MREM)
        pltpu.sync_copy(acc.at[rsl], out_hbm.at[cid].at[rsl])


BM = 1000


def _combine_body(p_ref, obs_ref, gtab_ref, loss_ref):
    i = pl.program_id(0)
    g = p_ref[0] + p_ref[1] - obs_ref[...]
    gtab_ref[...] = g

    @pl.when(i == 0)
    def _():
        loss_ref[0, 0] = 0.0

    loss_ref[0, 0] += 0.5 * jnp.sum(g * g)


_combine = pl.pallas_call(
    _combine_body,
    grid=(M // BM,),
    in_specs=[
        pl.BlockSpec((2, BM, D), lambda i: (0, i, 0)),
        pl.BlockSpec((BM, D), lambda i: (i, 0)),
    ],
    out_specs=[
        pl.BlockSpec((BM, D), lambda i: (i, 0)),
        pl.BlockSpec(memory_space=pltpu.SMEM),
    ],
    out_shape=[
        jax.ShapeDtypeStruct((M, D), jnp.float32),
        jax.ShapeDtypeStruct((1, 1), jnp.float32),
    ],
)


@functools.partial(
    pl.kernel,
    out_type=jax.ShapeDtypeStruct((N, D), jnp.float32),
    mesh=_mesh,
    scratch_types=[
        pltpu.VMEM((CPT, CH), jnp.int32),
        pltpu.VMEM((CH, D), jnp.float32),
        pltpu.SemaphoreType.DMA,
    ],
)
def _gather_grad(gtab_hbm, idx_hbm, out_hbm, idxblk, buf, sem):
    cid = lax.axis_index("c")
    sid = lax.axis_index("s")
    wid = sid * NC + cid
    row0 = wid * ROWS_PT
    pltpu.sync_copy(idx_hbm.at[wid], idxblk)

    def body(j, carry):
        pltpu.async_copy(gtab_hbm.at[idxblk.at[j]], buf, sem).wait()
        pltpu.sync_copy(buf, out_hbm.at[pl.ds(row0 + j * CH, CH)])
        return carry

    lax.fori_loop(0, CPT, body, 0)


def kernel(mu_0, obs, idx):
    idx3d = idx.reshape(NW, CPT, CH)
    zeros = jnp.zeros((M, D), jnp.float32)
    partials = _scatter_partials(mu_0, idx3d, zeros)
    gtab, loss2d = _combine(partials, obs)
    grad = _gather_grad(gtab, idx3d)
    return loss2d[0, 0], grad


# speedup vs baseline: 4.2665x; 1.7425x over previous
import functools

import jax
import jax.numpy as jnp
from jax import lax
from jax.experimental import pallas as pl
from jax.experimental.pallas import tpu as pltpu
from jax.experimental.pallas import tpu_sc as plsc

N = 320000
M = 10000
D = 128

NC = 2
NS = 16
NW = NC * NS

ROWS_PT = N // NW
NBUF = 5
SCH = 80
SCPT = ROWS_PT // SCH
SNBUF = 3
SNG = SCPT // SNBUF
SREM = SCPT - SNG * SNBUF
CH = 80
CPT = ROWS_PT // CH
NG = CPT // NBUF
MSA = 624
MREM = M - NS * MSA

_mesh = plsc.VectorSubcoreMesh(core_axis_name="c", subcore_axis_name="s")


@functools.partial(
    pl.kernel,
    out_type=jax.ShapeDtypeStruct((NC, M, D), jnp.float32),
    mesh=_mesh,
    scratch_types=[
        pltpu.VMEM((SCPT, SCH), jnp.int32),
        pltpu.VMEM((SNBUF, SCH, D), jnp.float32),
        pltpu.VMEM_SHARED((M, D), jnp.float32),
    ]
    + [pltpu.SemaphoreType.DMA] * SNBUF,
)
def _scatter_partials(mu_hbm, idx_hbm, zeros_hbm, out_hbm, idxblk, bufs, acc,
                      *lsems):
    cid = lax.axis_index("c")
    sid = lax.axis_index("s")
    wid = sid * NC + cid
    row0 = wid * ROWS_PT
    pltpu.sync_copy(idx_hbm.at[wid], idxblk)
    msl = pl.ds(sid * MSA, MSA)
    pltpu.sync_copy(zeros_hbm.at[msl], acc.at[msl])

    @pl.when(sid == 0)
    def _():
        rsl = pl.ds(NS * MSA, MREM)
        pltpu.sync_copy(zeros_hbm.at[rsl], acc.at[rsl])

    plsc.subcore_barrier()

    for b in range(SNBUF):
        pltpu.async_copy(mu_hbm.at[pl.ds(row0 + b * SCH, SCH)], bufs.at[b],
                         lsems[b])

    def body(g0, carry):
        descs = []
        for b in range(SNBUF):
            j = g0 * SNBUF + b
            pltpu.make_async_copy(mu_hbm.at[pl.ds(0, SCH)], bufs.at[b],
                                  lsems[b]).wait()
            descs.append(
                pltpu.async_copy(bufs.at[b], acc.at[idxblk.at[j]],
                                 lsems[b], add=True))

        for b in range(SNBUF):
            jn = (g0 + 1) * SNBUF + b
            descs[b].wait()

            @pl.when(jn < SCPT)
            def _():
                pltpu.async_copy(mu_hbm.at[pl.ds(row0 + jn * SCH, SCH)],
                                 bufs.at[b], lsems[b])

        return carry

    lax.fori_loop(0, SNG, body, 0)
    rdescs = []
    for r in range(SREM):
        j = SNG * SNBUF + r
        pltpu.make_async_copy(mu_hbm.at[pl.ds(0, SCH)], bufs.at[r],
                              lsems[r]).wait()
        rdescs.append(
            pltpu.async_copy(bufs.at[r], acc.at[idxblk.at[j]], lsems[r],
                             add=True))
    for d in rdescs:
        d.wait()
    plsc.subcore_barrier()
    pltpu.sync_copy(acc.at[msl], out_hbm.at[cid].at[msl])

    @pl.when(sid == 0)
    def _():
        rsl = pl.ds(NS * MSA, MREM)
        pltpu.sync_copy(acc.at[rsl], out_hbm.at[cid].at[rsl])


BM = 1000


def _combine_body(p_ref, obs_ref, gtab_ref, loss_ref):
    i = pl.program_id(0)
    g = p_ref[0] + p_ref[1] - obs_ref[...]
    gtab_ref[...] = g

    @pl.when(i == 0)
    def _():
        loss_ref[0, 0] = 0.0

    loss_ref[0, 0] += 0.5 * jnp.sum(g * g)


_combine = pl.pallas_call(
    _combine_body,
    grid=(M // BM,),
    in_specs=[
        pl.BlockSpec((2, BM, D), lambda i: (0, i, 0)),
        pl.BlockSpec((BM, D), lambda i: (i, 0)),
    ],
    out_specs=[
        pl.BlockSpec((BM, D), lambda i: (i, 0)),
        pl.BlockSpec(memory_space=pltpu.SMEM),
    ],
    out_shape=[
        jax.ShapeDtypeStruct((M, D), jnp.float32),
        jax.ShapeDtypeStruct((1, 1), jnp.float32),
    ],
)


@functools.partial(
    pl.kernel,
    out_type=jax.ShapeDtypeStruct((N, D), jnp.float32),
    mesh=_mesh,
    scratch_types=[
        pltpu.VMEM((CPT, CH), jnp.int32),
        pltpu.VMEM((NBUF, CH, D), jnp.float32),
    ]
    + [pltpu.SemaphoreType.DMA] * (2 * NBUF),
)
def _gather_grad(gtab_hbm, idx_hbm, out_hbm, idxblk, bufs, *sems):
    gsems, wsems = sems[:NBUF], sems[NBUF:]
    cid = lax.axis_index("c")
    sid = lax.axis_index("s")
    wid = sid * NC + cid
    row0 = wid * ROWS_PT
    pltpu.sync_copy(idx_hbm.at[wid], idxblk)

    for b in range(NBUF):
        pltpu.async_copy(gtab_hbm.at[idxblk.at[b]], bufs.at[b], gsems[b])

    def body(g0, carry):
        descs = []
        for b in range(NBUF):
            j = g0 * NBUF + b
            pltpu.make_async_copy(gtab_hbm.at[pl.ds(0, CH)], bufs.at[b],
                                  gsems[b]).wait()
            descs.append(
                pltpu.async_copy(bufs.at[b],
                                 out_hbm.at[pl.ds(row0 + j * CH, CH)],
                                 wsems[b]))

        @pl.when(g0 < NG - 1)
        def _():
            for b in range(NBUF):
                j = (g0 + 1) * NBUF + b
                descs[b].wait()
                pltpu.async_copy(gtab_hbm.at[idxblk.at[j]], bufs.at[b],
                                 gsems[b])

        @pl.when(g0 == NG - 1)
        def _():
            for d in descs:
                d.wait()

        return carry

    lax.fori_loop(0, NG, body, 0)


def kernel(mu_0, obs, idx):
    idx_s = idx.reshape(NW, SCPT, SCH)
    idx_g = idx.reshape(NW, CPT, CH)
    zeros = jnp.zeros((M, D), jnp.float32)
    partials = _scatter_partials(mu_0, idx_s, zeros)
    gtab, loss2d = _combine(partials, obs)
    grad = _gather_grad(gtab, idx_g)
    return loss2d[0, 0], grad


# speedup vs baseline: 4.4692x; 1.0475x over previous
import functools

import jax
import jax.numpy as jnp
from jax import lax
from jax.experimental import pallas as pl
from jax.experimental.pallas import tpu as pltpu
from jax.experimental.pallas import tpu_sc as plsc

N = 320000
M = 10000
D = 128

NC = 2
NS = 16
NW = NC * NS

ROWS_PT = N // NW
NBUF = 5
SCH = 80
SCPT = ROWS_PT // SCH
SNBUF = 3
SNG = SCPT // SNBUF
SREM = SCPT - SNG * SNBUF
CH = 80
CPT = ROWS_PT // CH
GNBUF = 10
NG = CPT // GNBUF
GREM = CPT - NG * GNBUF
MSA = 624
MREM = M - NS * MSA

_mesh = plsc.VectorSubcoreMesh(core_axis_name="c", subcore_axis_name="s")


@functools.partial(
    pl.kernel,
    out_type=jax.ShapeDtypeStruct((NC, M, D), jnp.float32),
    mesh=_mesh,
    scratch_types=[
        pltpu.VMEM((SCPT, SCH), jnp.int32),
        pltpu.VMEM((SNBUF, SCH, D), jnp.float32),
        pltpu.VMEM_SHARED((M, D), jnp.float32),
    ]
    + [pltpu.SemaphoreType.DMA] * SNBUF,
)
def _scatter_partials(mu_hbm, idx_hbm, zeros_hbm, out_hbm, idxblk, bufs, acc,
                      *lsems):
    cid = lax.axis_index("c")
    sid = lax.axis_index("s")
    wid = sid * NC + cid
    row0 = wid * ROWS_PT
    pltpu.sync_copy(idx_hbm.at[wid], idxblk)
    msl = pl.ds(sid * MSA, MSA)
    pltpu.sync_copy(zeros_hbm.at[msl], acc.at[msl])

    @pl.when(sid == 0)
    def _():
        rsl = pl.ds(NS * MSA, MREM)
        pltpu.sync_copy(zeros_hbm.at[rsl], acc.at[rsl])

    plsc.subcore_barrier()

    for b in range(SNBUF):
        pltpu.async_copy(mu_hbm.at[pl.ds(row0 + b * SCH, SCH)], bufs.at[b],
                         lsems[b])

    def body(g0, carry):
        descs = []
        for b in range(SNBUF):
            j = g0 * SNBUF + b
            pltpu.make_async_copy(mu_hbm.at[pl.ds(0, SCH)], bufs.at[b],
                                  lsems[b]).wait()
            descs.append(
                pltpu.async_copy(bufs.at[b], acc.at[idxblk.at[j]],
                                 lsems[b], add=True))

        for b in range(SNBUF):
            jn = (g0 + 1) * SNBUF + b
            descs[b].wait()

            @pl.when(jn < SCPT)
            def _():
                pltpu.async_copy(mu_hbm.at[pl.ds(row0 + jn * SCH, SCH)],
                                 bufs.at[b], lsems[b])

        return carry

    lax.fori_loop(0, SNG, body, 0)
    rdescs = []
    for r in range(SREM):
        j = SNG * SNBUF + r
        pltpu.make_async_copy(mu_hbm.at[pl.ds(0, SCH)], bufs.at[r],
                              lsems[r]).wait()
        rdescs.append(
            pltpu.async_copy(bufs.at[r], acc.at[idxblk.at[j]], lsems[r],
                             add=True))
    for d in rdescs:
        d.wait()
    plsc.subcore_barrier()
    pltpu.sync_copy(acc.at[msl], out_hbm.at[cid].at[msl])

    @pl.when(sid == 0)
    def _():
        rsl = pl.ds(NS * MSA, MREM)
        pltpu.sync_copy(acc.at[rsl], out_hbm.at[cid].at[rsl])


BM = 1000


def _combine_body(p_ref, obs_ref, gtab_ref, loss_ref):
    i = pl.program_id(0)
    g = p_ref[0] + p_ref[1] - obs_ref[...]
    gtab_ref[...] = g

    @pl.when(i == 0)
    def _():
        loss_ref[0, 0] = 0.0

    loss_ref[0, 0] += 0.5 * jnp.sum(g * g)


_combine = pl.pallas_call(
    _combine_body,
    grid=(M // BM,),
    in_specs=[
        pl.BlockSpec((2, BM, D), lambda i: (0, i, 0)),
        pl.BlockSpec((BM, D), lambda i: (i, 0)),
    ],
    out_specs=[
        pl.BlockSpec((BM, D), lambda i: (i, 0)),
        pl.BlockSpec(memory_space=pltpu.SMEM),
    ],
    out_shape=[
        jax.ShapeDtypeStruct((M, D), jnp.float32),
        jax.ShapeDtypeStruct((1, 1), jnp.float32),
    ],
)


@functools.partial(
    pl.kernel,
    out_type=jax.ShapeDtypeStruct((N, D), jnp.float32),
    mesh=_mesh,
    scratch_types=[
        pltpu.VMEM((CPT, CH), jnp.int32),
        pltpu.VMEM((GNBUF, CH, D), jnp.float32),
    ]
    + [pltpu.SemaphoreType.DMA] * (2 * GNBUF),
)
def _gather_grad(gtab_hbm, idx_hbm, out_hbm, idxblk, bufs, *sems):
    gsems, wsems = sems[:GNBUF], sems[GNBUF:]
    cid = lax.axis_index("c")
    sid = lax.axis_index("s")
    wid = sid * NC + cid
    row0 = wid * ROWS_PT
    pltpu.sync_copy(idx_hbm.at[wid], idxblk)

    for b in range(GNBUF):
        pltpu.async_copy(gtab_hbm.at[idxblk.at[b]], bufs.at[b], gsems[b])

    def body(g0, carry):
        descs = []
        for b in range(GNBUF):
            j = g0 * GNBUF + b
            pltpu.make_async_copy(gtab_hbm.at[pl.ds(0, CH)], bufs.at[b],
                                  gsems[b]).wait()
            descs.append(
                pltpu.async_copy(bufs.at[b],
                                 out_hbm.at[pl.ds(row0 + j * CH, CH)],
                                 wsems[b]))

        for b in range(GNBUF):
            jn = (g0 + 1) * GNBUF + b
            descs[b].wait()

            @pl.when(jn < CPT)
            def _():
                pltpu.async_copy(gtab_hbm.at[idxblk.at[jn]], bufs.at[b],
                                 gsems[b])

        return carry

    lax.fori_loop(0, NG, body, 0)
    rdescs = []
    for r in range(GREM):
        j = NG * GNBUF + r
        pltpu.make_async_copy(gtab_hbm.at[pl.ds(0, CH)], bufs.at[r],
                              gsems[r]).wait()
        rdescs.append(
            pltpu.async_copy(bufs.at[r], out_hbm.at[pl.ds(row0 + j * CH, CH)],
                             wsems[r]))
    for d in rdescs:
        d.wait()


def kernel(mu_0, obs, idx):
    idx_s = idx.reshape(NW, SCPT, SCH)
    idx_g = idx.reshape(NW, CPT, CH)
    zeros = jnp.zeros((M, D), jnp.float32)
    partials = _scatter_partials(mu_0, idx_s, zeros)
    gtab, loss2d = _combine(partials, obs)
    grad = _gather_grad(gtab, idx_g)
    return loss2d[0, 0], grad


# speedup vs baseline: 8.3099x; 1.8594x over previous
import functools

import jax
import jax.numpy as jnp
from jax import lax
from jax.experimental import pallas as pl
from jax.experimental.pallas import tpu as pltpu
from jax.experimental.pallas import tpu_sc as plsc

N = 320000
M = 10000
D = 128

NC = 2
NS = 16
NW = NC * NS

ROWS_PT = N // NW
NBUF = 5
SCH = 80
SCPT = ROWS_PT // SCH
SNBUF = 3
SNG = SCPT // SNBUF
SREM = SCPT - SNG * SNBUF
CH = 80
CPT = ROWS_PT // CH
GNBUF = 4
NG = CPT // GNBUF
GREM = CPT - NG * GNBUF
MSA = 624
MREM = M - NS * MSA

_mesh = plsc.VectorSubcoreMesh(core_axis_name="c", subcore_axis_name="s")


@functools.partial(
    pl.kernel,
    out_type=jax.ShapeDtypeStruct((NC, M, D), jnp.float32),
    mesh=_mesh,
    scratch_types=[
        pltpu.VMEM((SCPT, SCH), jnp.int32),
        pltpu.VMEM((SNBUF, SCH, D), jnp.float32),
        pltpu.VMEM_SHARED((M, D), jnp.float32),
    ]
    + [pltpu.SemaphoreType.DMA] * SNBUF,
)
def _scatter_partials(mu_hbm, idx_hbm, zeros_hbm, out_hbm, idxblk, bufs, acc,
                      *lsems):
    cid = lax.axis_index("c")
    sid = lax.axis_index("s")
    wid = sid * NC + cid
    row0 = wid * ROWS_PT
    pltpu.sync_copy(idx_hbm.at[wid], idxblk)
    msl = pl.ds(sid * MSA, MSA)
    pltpu.sync_copy(zeros_hbm.at[msl], acc.at[msl])

    @pl.when(sid == 0)
    def _():
        rsl = pl.ds(NS * MSA, MREM)
        pltpu.sync_copy(zeros_hbm.at[rsl], acc.at[rsl])

    plsc.subcore_barrier()

    for b in range(SNBUF):
        pltpu.async_copy(mu_hbm.at[pl.ds(row0 + b * SCH, SCH)], bufs.at[b],
                         lsems[b])

    def body(g0, carry):
        descs = []
        for b in range(SNBUF):
            j = g0 * SNBUF + b
            pltpu.make_async_copy(mu_hbm.at[pl.ds(0, SCH)], bufs.at[b],
                                  lsems[b]).wait()
            descs.append(
                pltpu.async_copy(bufs.at[b], acc.at[idxblk.at[j]],
                                 lsems[b], add=True))

        for b in range(SNBUF):
            jn = (g0 + 1) * SNBUF + b
            descs[b].wait()

            @pl.when(jn < SCPT)
            def _():
                pltpu.async_copy(mu_hbm.at[pl.ds(row0 + jn * SCH, SCH)],
                                 bufs.at[b], lsems[b])

        return carry

    lax.fori_loop(0, SNG, body, 0)
    rdescs = []
    for r in range(SREM):
        j = SNG * SNBUF + r
        pltpu.make_async_copy(mu_hbm.at[pl.ds(0, SCH)], bufs.at[r],
                              lsems[r]).wait()
        rdescs.append(
            pltpu.async_copy(bufs.at[r], acc.at[idxblk.at[j]], lsems[r],
                             add=True))
    for d in rdescs:
        d.wait()
    plsc.subcore_barrier()
    pltpu.sync_copy(acc.at[msl], out_hbm.at[cid].at[msl])

    @pl.when(sid == 0)
    def _():
        rsl = pl.ds(NS * MSA, MREM)
        pltpu.sync_copy(acc.at[rsl], out_hbm.at[cid].at[rsl])


BM = 1000


def _combine_body(p_ref, obs_ref, gtab_ref, loss_ref):
    i = pl.program_id(0)
    g = p_ref[0] + p_ref[1] - obs_ref[...]
    gtab_ref[...] = g

    @pl.when(i == 0)
    def _():
        loss_ref[0, 0] = 0.0

    loss_ref[0, 0] += 0.5 * jnp.sum(g * g)


_combine = pl.pallas_call(
    _combine_body,
    grid=(M // BM,),
    in_specs=[
        pl.BlockSpec((2, BM, D), lambda i: (0, i, 0)),
        pl.BlockSpec((BM, D), lambda i: (i, 0)),
    ],
    out_specs=[
        pl.BlockSpec((BM, D), lambda i: (i, 0)),
        pl.BlockSpec(memory_space=pltpu.SMEM),
    ],
    out_shape=[
        jax.ShapeDtypeStruct((M, D), jnp.float32),
        jax.ShapeDtypeStruct((1, 1), jnp.float32),
    ],
)


@functools.partial(
    pl.kernel,
    out_type=jax.ShapeDtypeStruct((N, D), jnp.float32),
    mesh=_mesh,
    scratch_types=[
        pltpu.VMEM((ROWS_PT,), jnp.int32),
        pltpu.VMEM((GNBUF, CH, D), jnp.float32),
        pltpu.VMEM_SHARED((M, D), jnp.float32),
    ]
    + [pltpu.SemaphoreType.DMA] * (2 * GNBUF),
)
def _gather_grad(gtab_hbm, idx_hbm, out_hbm, idxblk, bufs, gsh, *sems):
    gsems, wsems = sems[:GNBUF], sems[GNBUF:]
    cid = lax.axis_index("c")
    sid = lax.axis_index("s")
    wid = sid * NC + cid
    row0 = wid * ROWS_PT
    pltpu.sync_copy(idx_hbm.at[pl.ds(row0, ROWS_PT)], idxblk)
    msl = pl.ds(sid * MSA, MSA)
    pltpu.sync_copy(gtab_hbm.at[msl], gsh.at[msl])

    @pl.when(sid == 0)
    def _():
        rsl = pl.ds(NS * MSA, MREM)
        pltpu.sync_copy(gtab_hbm.at[rsl], gsh.at[rsl])

    plsc.subcore_barrier()

    for b in range(GNBUF):
        pltpu.async_copy(gsh.at[idxblk.at[pl.ds(b * CH, CH)]], bufs.at[b],
                         gsems[b])

    def body(g0, carry):
        descs = []
        for b in range(GNBUF):
            j = g0 * GNBUF + b
            pltpu.make_async_copy(gtab_hbm.at[pl.ds(0, CH)], bufs.at[b],
                                  gsems[b]).wait()
            descs.append(
                pltpu.async_copy(bufs.at[b],
                                 out_hbm.at[pl.ds(row0 + j * CH, CH)],
                                 wsems[b]))

        for b in range(GNBUF):
            jn = (g0 + 1) * GNBUF + b
            descs[b].wait()

            @pl.when(jn < CPT)
            def _():
                pltpu.async_copy(gsh.at[idxblk.at[pl.ds(jn * CH, CH)]],
                                 bufs.at[b], gsems[b])

        return carry

    lax.fori_loop(0, NG, body, 0)
    rdescs = []
    for r in range(GREM):
        j = NG * GNBUF + r
        pltpu.make_async_copy(gtab_hbm.at[pl.ds(0, CH)], bufs.at[r],
                              gsems[r]).wait()
        rdescs.append(
            pltpu.async_copy(bufs.at[r], out_hbm.at[pl.ds(row0 + j * CH, CH)],
                             wsems[r]))
    for d in rdescs:
        d.wait()


def kernel(mu_0, obs, idx):
    idx_s = idx.reshape(NW, SCPT, SCH)
    zeros = jnp.zeros((M, D), jnp.float32)
    partials = _scatter_partials(mu_0, idx_s, zeros)
    gtab, loss2d = _combine(partials, obs)
    grad = _gather_grad(gtab, idx)
    return loss2d[0, 0], grad


# speedup vs baseline: 8.6727x; 1.0437x over previous
import functools

import jax
import jax.numpy as jnp
from jax import lax
from jax.experimental import pallas as pl
from jax.experimental.pallas import tpu as pltpu
from jax.experimental.pallas import tpu_sc as plsc

N = 320000
M = 10000
D = 128

NC = 2
NS = 16
NW = NC * NS

ROWS_PT = N // NW
NBUF = 5
SCH = 80
SCPT = ROWS_PT // SCH
SHALF = 64
SNBUF = 4
SNG = SCPT // SNBUF
SREM = SCPT - SNG * SNBUF
CH = 80
CPT = ROWS_PT // CH
GNBUF = 4
NG = CPT // GNBUF
GREM = CPT - NG * GNBUF
MSA = 624
MREM = M - NS * MSA

_mesh = plsc.VectorSubcoreMesh(core_axis_name="c", subcore_axis_name="s")


@functools.partial(
    pl.kernel,
    out_type=jax.ShapeDtypeStruct((NC, M, D), jnp.float32),
    mesh=_mesh,
    scratch_types=[
        pltpu.VMEM((SHALF, SCH), jnp.int32),
        pltpu.VMEM((SNBUF, SCH, D), jnp.float32),
        pltpu.VMEM_SHARED((M, D), jnp.float32),
    ]
    + [pltpu.SemaphoreType.DMA] * SNBUF,
)
def _scatter_partials(mu_hbm, idx_hbm, zeros_hbm, out_hbm, idxblk, bufs, acc,
                      *lsems):
    cid = lax.axis_index("c")
    sid = lax.axis_index("s")
    wid = sid * NC + cid
    row0 = wid * ROWS_PT
    pltpu.sync_copy(idx_hbm.at[wid].at[pl.ds(0, SHALF)], idxblk)
    msl = pl.ds(sid * MSA, MSA)
    pltpu.sync_copy(zeros_hbm.at[msl], acc.at[msl])

    @pl.when(sid == 0)
    def _():
        rsl = pl.ds(NS * MSA, MREM)
        pltpu.sync_copy(zeros_hbm.at[rsl], acc.at[rsl])

    plsc.subcore_barrier()

    for b in range(SNBUF):
        pltpu.async_copy(mu_hbm.at[pl.ds(row0 + b * SCH, SCH)], bufs.at[b],
                         lsems[b])

    def body(g0, carry):
        @pl.when(g0 == SHALF // SNBUF)
        def _():
            pltpu.sync_copy(idx_hbm.at[wid].at[pl.ds(SHALF, SHALF)], idxblk)

        descs = []
        for b in range(SNBUF):
            j = g0 * SNBUF + b
            jloc = lax.select(j >= SHALF, j - SHALF, j)
            pltpu.make_async_copy(mu_hbm.at[pl.ds(0, SCH)], bufs.at[b],
                                  lsems[b]).wait()
            descs.append(
                pltpu.async_copy(bufs.at[b], acc.at[idxblk.at[jloc]],
                                 lsems[b], add=True))

        for b in range(SNBUF):
            jn = (g0 + 1) * SNBUF + b
            descs[b].wait()

            @pl.when(jn < SCPT)
            def _():
                pltpu.async_copy(mu_hbm.at[pl.ds(row0 + jn * SCH, SCH)],
                                 bufs.at[b], lsems[b])

        return carry

    lax.fori_loop(0, SNG, body, 0)
    rdescs = []
    for r in range(SREM):
        j = SNG * SNBUF + r
        pltpu.make_async_copy(mu_hbm.at[pl.ds(0, SCH)], bufs.at[r],
                              lsems[r]).wait()
        rdescs.append(
            pltpu.async_copy(bufs.at[r], acc.at[idxblk.at[j - SHALF]],
                             lsems[r], add=True))
    for d in rdescs:
        d.wait()
    plsc.subcore_barrier()
    pltpu.sync_copy(acc.at[msl], out_hbm.at[cid].at[msl])

    @pl.when(sid == 0)
    def _():
        rsl = pl.ds(NS * MSA, MREM)
        pltpu.sync_copy(acc.at[rsl], out_hbm.at[cid].at[rsl])


BM = 1000


def _combine_body(p_ref, obs_ref, gtab_ref, loss_ref):
    i = pl.program_id(0)
    g = p_ref[0] + p_ref[1] - obs_ref[...]
    gtab_ref[...] = g

    @pl.when(i == 0)
    def _():
        loss_ref[0, 0] = 0.0

    loss_ref[0, 0] += 0.5 * jnp.sum(g * g)


_combine = pl.pallas_call(
    _combine_body,
    grid=(M // BM,),
    in_specs=[
        pl.BlockSpec((2, BM, D), lambda i: (0, i, 0)),
        pl.BlockSpec((BM, D), lambda i: (i, 0)),
    ],
    out_specs=[
        pl.BlockSpec((BM, D), lambda i: (i, 0)),
        pl.BlockSpec(memory_space=pltpu.SMEM),
    ],
    out_shape=[
        jax.ShapeDtypeStruct((M, D), jnp.float32),
        jax.ShapeDtypeStruct((1, 1), jnp.float32),
    ],
)


@functools.partial(
    pl.kernel,
    out_type=jax.ShapeDtypeStruct((N, D), jnp.float32),
    mesh=_mesh,
    scratch_types=[
        pltpu.VMEM((ROWS_PT,), jnp.int32),
        pltpu.VMEM((GNBUF, CH, D), jnp.float32),
        pltpu.VMEM_SHARED((M, D), jnp.float32),
    ]
    + [pltpu.SemaphoreType.DMA] * (2 * GNBUF),
)
def _gather_grad(gtab_hbm, idx_hbm, out_hbm, idxblk, bufs, gsh, *sems):
    gsems, wsems = sems[:GNBUF], sems[GNBUF:]
    cid = lax.axis_index("c")
    sid = lax.axis_index("s")
    wid = sid * NC + cid
    row0 = wid * ROWS_PT
    pltpu.sync_copy(idx_hbm.at[pl.ds(row0, ROWS_PT)], idxblk)
    msl = pl.ds(sid * MSA, MSA)
    pltpu.sync_copy(gtab_hbm.at[msl], gsh.at[msl])

    @pl.when(sid == 0)
    def _():
        rsl = pl.ds(NS * MSA, MREM)
        pltpu.sync_copy(gtab_hbm.at[rsl], gsh.at[rsl])

    plsc.subcore_barrier()

    for b in range(GNBUF):
        pltpu.async_copy(gsh.at[idxblk.at[pl.ds(b * CH, CH)]], bufs.at[b],
                         gsems[b])

    def body(g0, carry):
        descs = []
        for b in range(GNBUF):
            j = g0 * GNBUF + b
            pltpu.make_async_copy(gtab_hbm.at[pl.ds(0, CH)], bufs.at[b],
                                  gsems[b]).wait()
            descs.append(
                pltpu.async_copy(bufs.at[b],
                                 out_hbm.at[pl.ds(row0 + j * CH, CH)],
                                 wsems[b]))

        for b in range(GNBUF):
            jn = (g0 + 1) * GNBUF + b
            descs[b].wait()

            @pl.when(jn < CPT)
            def _():
                pltpu.async_copy(gsh.at[idxblk.at[pl.ds(jn * CH, CH)]],
                                 bufs.at[b], gsems[b])

        return carry

    lax.fori_loop(0, NG, body, 0)
    rdescs = []
    for r in range(GREM):
        j = NG * GNBUF + r
        pltpu.make_async_copy(gtab_hbm.at[pl.ds(0, CH)], bufs.at[r],
                              gsems[r]).wait()
        rdescs.append(
            pltpu.async_copy(bufs.at[r], out_hbm.at[pl.ds(row0 + j * CH, CH)],
                             wsems[r]))
    for d in rdescs:
        d.wait()


def kernel(mu_0, obs, idx):
    idx_s = jnp.pad(idx.reshape(NW, SCPT, SCH),
                    ((0, 0), (0, 2 * SHALF - SCPT), (0, 0)))
    zeros = jnp.zeros((M, D), jnp.float32)
    partials = _scatter_partials(mu_0, idx_s, zeros)
    gtab, loss2d = _combine(partials, obs)
    grad = _gather_grad(gtab, idx)
    return loss2d[0, 0], grad


# speedup vs baseline: 8.7770x; 1.0120x over previous
import functools

import jax
import jax.numpy as jnp
from jax import lax
from jax.experimental import pallas as pl
from jax.experimental.pallas import tpu as pltpu
from jax.experimental.pallas import tpu_sc as plsc

N = 320000
M = 10000
D = 128

NC = 2
NS = 16
NW = NC * NS

ROWS_PT = N // NW
NBUF = 5
SCH = 80
SCPT = ROWS_PT // SCH
SHALF = 64
SNBUF = 4
SNG = SCPT // SNBUF
SREM = SCPT - SNG * SNBUF
CH = 80
CPT = ROWS_PT // CH
GNBUF = 4
NG = CPT // GNBUF
GREM = CPT - NG * GNBUF
MSA = 624
MREM = M - NS * MSA

_mesh = plsc.VectorSubcoreMesh(core_axis_name="c", subcore_axis_name="s")


@functools.partial(
    pl.kernel,
    out_type=jax.ShapeDtypeStruct((NC, M, D), jnp.float32),
    mesh=_mesh,
    scratch_types=[
        pltpu.VMEM((SHALF, SCH), jnp.int32),
        pltpu.VMEM((SNBUF, SCH, D), jnp.float32),
        pltpu.VMEM_SHARED((M, D), jnp.float32),
    ]
    + [pltpu.SemaphoreType.DMA] * SNBUF,
)
def _scatter_partials(mu_hbm, idx_hbm, zeros_hbm, out_hbm, idxblk, bufs, acc,
                      *lsems):
    cid = lax.axis_index("c")
    sid = lax.axis_index("s")
    wid = sid * NC + cid
    row0 = wid * ROWS_PT
    pltpu.sync_copy(idx_hbm.at[wid].at[pl.ds(0, SHALF)], idxblk)
    msl = pl.ds(sid * MSA, MSA)
    pltpu.sync_copy(zeros_hbm.at[msl], acc.at[msl])

    @pl.when(sid == 0)
    def _():
        rsl = pl.ds(NS * MSA, MREM)
        pltpu.sync_copy(zeros_hbm.at[rsl], acc.at[rsl])

    plsc.subcore_barrier()

    for b in range(SNBUF):
        pltpu.async_copy(mu_hbm.at[pl.ds(row0 + b * SCH, SCH)], bufs.at[b],
                         lsems[b])

    def body(g0, carry):
        @pl.when(g0 == SHALF // SNBUF)
        def _():
            pltpu.sync_copy(idx_hbm.at[wid].at[pl.ds(SHALF, SHALF)], idxblk)

        descs = []
        for b in range(SNBUF):
            j = g0 * SNBUF + b
            jloc = lax.select(j >= SHALF, j - SHALF, j)
            pltpu.make_async_copy(mu_hbm.at[pl.ds(0, SCH)], bufs.at[b],
                                  lsems[b]).wait()
            descs.append(
                pltpu.async_copy(bufs.at[b], acc.at[idxblk.at[jloc]],
                                 lsems[b], add=True))

        for b in range(SNBUF):
            jn = (g0 + 1) * SNBUF + b
            descs[b].wait()

            @pl.when(jn < SCPT)
            def _():
                pltpu.async_copy(mu_hbm.at[pl.ds(row0 + jn * SCH, SCH)],
                                 bufs.at[b], lsems[b])

        return carry

    lax.fori_loop(0, SNG, body, 0)
    rdescs = []
    for r in range(SREM):
        j = SNG * SNBUF + r
        pltpu.make_async_copy(mu_hbm.at[pl.ds(0, SCH)], bufs.at[r],
                              lsems[r]).wait()
        rdescs.append(
            pltpu.async_copy(bufs.at[r], acc.at[idxblk.at[j - SHALF]],
                             lsems[r], add=True))
    for d in rdescs:
        d.wait()
    plsc.subcore_barrier()
    pltpu.sync_copy(acc.at[msl], out_hbm.at[cid].at[msl])

    @pl.when(sid == 0)
    def _():
        rsl = pl.ds(NS * MSA, MREM)
        pltpu.sync_copy(acc.at[rsl], out_hbm.at[cid].at[rsl])


BM = 2000


def _combine_body(p_ref, obs_ref, gtab_ref, loss_ref):
    i = pl.program_id(0)
    g = p_ref[0] + p_ref[1] - obs_ref[...]
    gtab_ref[...] = g

    @pl.when(i == 0)
    def _():
        loss_ref[0, 0] = 0.0

    loss_ref[0, 0] += 0.5 * jnp.sum(g * g)


_combine = pl.pallas_call(
    _combine_body,
    grid=(M // BM,),
    in_specs=[
        pl.BlockSpec((2, BM, D), lambda i: (0, i, 0)),
        pl.BlockSpec((BM, D), lambda i: (i, 0)),
    ],
    out_specs=[
        pl.BlockSpec((BM, D), lambda i: (i, 0)),
        pl.BlockSpec(memory_space=pltpu.SMEM),
    ],
    out_shape=[
        jax.ShapeDtypeStruct((M, D), jnp.float32),
        jax.ShapeDtypeStruct((1, 1), jnp.float32),
    ],
)


@functools.partial(
    pl.kernel,
    out_type=jax.ShapeDtypeStruct((N, D), jnp.float32),
    mesh=_mesh,
    scratch_types=[
        pltpu.VMEM((ROWS_PT,), jnp.int32),
        pltpu.VMEM((GNBUF, CH, D), jnp.float32),
        pltpu.VMEM_SHARED((M, D), jnp.float32),
    ]
    + [pltpu.SemaphoreType.DMA] * (2 * GNBUF),
)
def _gather_grad(gtab_hbm, idx_hbm, out_hbm, idxblk, bufs, gsh, *sems):
    gsems, wsems = sems[:GNBUF], sems[GNBUF:]
    cid = lax.axis_index("c")
    sid = lax.axis_index("s")
    wid = sid * NC + cid
    row0 = wid * ROWS_PT
    pltpu.sync_copy(idx_hbm.at[pl.ds(row0, ROWS_PT)], idxblk)
    msl = pl.ds(sid * MSA, MSA)
    pltpu.sync_copy(gtab_hbm.at[msl], gsh.at[msl])

    @pl.when(sid == 0)
    def _():
        rsl = pl.ds(NS * MSA, MREM)
        pltpu.sync_copy(gtab_hbm.at[rsl], gsh.at[rsl])

    plsc.subcore_barrier()

    for b in range(GNBUF):
        pltpu.async_copy(gsh.at[idxblk.at[pl.ds(b * CH, CH)]], bufs.at[b],
                         gsems[b])

    def body(g0, carry):
        descs = []
        for b in range(GNBUF):
            j = g0 * GNBUF + b
            pltpu.make_async_copy(gtab_hbm.at[pl.ds(0, CH)], bufs.at[b],
                                  gsems[b]).wait()
            descs.append(
                pltpu.async_copy(bufs.at[b],
                                 out_hbm.at[pl.ds(row0 + j * CH, CH)],
                                 wsems[b]))

        for b in range(GNBUF):
            jn = (g0 + 1) * GNBUF + b
            descs[b].wait()

            @pl.when(jn < CPT)
            def _():
                pltpu.async_copy(gsh.at[idxblk.at[pl.ds(jn * CH, CH)]],
                                 bufs.at[b], gsems[b])

        return carry

    lax.fori_loop(0, NG, body, 0)
    rdescs = []
    for r in range(GREM):
        j = NG * GNBUF + r
        pltpu.make_async_copy(gtab_hbm.at[pl.ds(0, CH)], bufs.at[r],
                              gsems[r]).wait()
        rdescs.append(
            pltpu.async_copy(bufs.at[r], out_hbm.at[pl.ds(row0 + j * CH, CH)],
                             wsems[r]))
    for d in rdescs:
        d.wait()


def kernel(mu_0, obs, idx):
    idx_s = jnp.pad(idx.reshape(NW, SCPT, SCH),
                    ((0, 0), (0, 2 * SHALF - SCPT), (0, 0)))
    zeros = jnp.zeros((M, D), jnp.float32)
    partials = _scatter_partials(mu_0, idx_s, zeros)
    gtab, loss2d = _combine(partials, obs)
    grad = _gather_grad(gtab, idx)
    return loss2d[0, 0], grad
